# baseline (device time: 231128 ns/iter reference)
import functools

import jax
import jax.numpy as jnp
from jax import lax
from jax.experimental import pallas as pl
from jax.experimental.pallas import tpu as pltpu

N_DEV = 4
SCALE = 0.08838834764831843
BLK = 64
PHASES = 4


def _allreduce_ring(partial):
    S, D = partial.shape

    def body(p_ref, out_ref, comm_ref, send_sems, recv_sems):
        my = lax.axis_index("i")
        left = (my - 1) % N_DEV
        right = (my + 1) % N_DEV

        barrier_sem = pltpu.get_barrier_semaphore()
        for nbr in [left, right]:
            pl.semaphore_signal(
                barrier_sem, inc=1,
                device_id=(nbr,), device_id_type=pl.DeviceIdType.MESH,
            )
        pl.semaphore_wait(barrier_sem, 2)

        comm_ref[0] = p_ref[...]
        acc = p_ref[...].astype(jnp.float32)

        for h in range(N_DEV - 1):
            rdma = pltpu.make_async_remote_copy(
                src_ref=comm_ref.at[h],
                dst_ref=comm_ref.at[h + 1],
                send_sem=send_sems.at[h],
                recv_sem=recv_sems.at[h],
                device_id=(right,),
                device_id_type=pl.DeviceIdType.MESH,
            )
            rdma.start()
            rdma.wait()
            acc = acc + comm_ref[h + 1].astype(jnp.float32)

        out_ref[...] = acc

    return pl.pallas_call(
        body,
        out_shape=jax.ShapeDtypeStruct((S, D), jnp.float32),
        in_specs=[pl.BlockSpec(memory_space=pltpu.VMEM)],
        out_specs=pl.BlockSpec(memory_space=pltpu.VMEM),
        scratch_shapes=[
            pltpu.VMEM((N_DEV, S, D), jnp.bfloat16),
            pltpu.SemaphoreType.DMA((N_DEV - 1,)),
            pltpu.SemaphoreType.DMA((N_DEV - 1,)),
        ],
        compiler_params=pltpu.CompilerParams(collective_id=0),
    )(partial)


def kernel(x, Wq, K_ext, V_ext, Wo):
    B, Sq, d_model = x.shape
    _, Skv, Hl, Dh = K_ext.shape
    Dl = Hl * Dh

    my = lax.axis_index("i")

    xb = x[0].astype(jnp.bfloat16)
    Wq_l = lax.dynamic_slice_in_dim(Wq, my * Dl, Dl, axis=1)
    Q = (xb @ Wq_l.astype(jnp.bfloat16)).reshape(Sq, Hl, Dh)
    K = K_ext[0].astype(jnp.bfloat16)
    V = V_ext[0].astype(jnp.bfloat16)

    n_blk = Sq // BLK
    m = n_blk // PHASES

    def group(t):
        t = t.reshape(m, PHASES, BLK, Hl, Dh)
        return t.transpose(1, 0, 2, 3, 4).reshape(PHASES, m * BLK, Hl, Dh)

    Qg, Kg, Vg = group(Q), group(K), group(V)
    scores = jnp.einsum(
        "pihd,pjhd->phij", Qg, Kg, preferred_element_type=jnp.float32
    ) * SCALE
    w = jax.nn.softmax(scores, axis=-1).astype(jnp.bfloat16)
    ctx = jnp.einsum("phij,pjhd->pihd", w, Vg)
    ctx = (
        ctx.reshape(PHASES, m, BLK, Hl, Dh)
        .transpose(1, 0, 2, 3, 4)
        .reshape(Sq, Dl)
    )

    Wo_l = lax.dynamic_slice_in_dim(Wo, my * Dl, Dl, axis=0)
    partial = jnp.dot(
        ctx, Wo_l.astype(jnp.bfloat16), preferred_element_type=jnp.float32
    ).astype(jnp.bfloat16)

    out = _allreduce_ring(partial)
    return out[None]


# device time: 128337 ns/iter; 1.8009x vs baseline; 1.8009x over previous
import jax
import jax.numpy as jnp
from jax import lax
from jax.experimental import pallas as pl
from jax.experimental.pallas import tpu as pltpu

N_DEV = 4
SCALE = 0.08838834764831843
BLK = 64
PHASES = 4


def _wo_allreduce(ctx, Wo_l):
    S, D = ctx.shape
    C = S // N_DEV
    H = C // 2

    def body(ctx_ref, wo_ref, out_ref, comm_ref, cw_send, cw_recv,
             ccw_send, ccw_recv):
        my = lax.axis_index("i")
        left = (my - 1) % N_DEV
        right = (my + 1) % N_DEV

        barrier_sem = pltpu.get_barrier_semaphore()
        for nbr in [left, right]:
            pl.semaphore_signal(
                barrier_sem, inc=1,
                device_id=(nbr,), device_id_type=pl.DeviceIdType.MESH,
            )
        pl.semaphore_wait(barrier_sem, 2)

        w = wo_ref[...]

        def pblock(row_start, n):
            a = ctx_ref[pl.ds(row_start, n), :]
            return jnp.dot(a, w, preferred_element_type=jnp.float32).astype(
                jnp.bfloat16
            )

        comm_ref[0] = pblock(my * C, C)

        for s in range(N_DEV - 1):
            cw = pltpu.make_async_remote_copy(
                src_ref=comm_ref.at[s, 0:H, :],
                dst_ref=comm_ref.at[s + 1, 0:H, :],
                send_sem=cw_send.at[s],
                recv_sem=cw_recv.at[s],
                device_id=(right,),
                device_id_type=pl.DeviceIdType.MESH,
            )
            ccw = pltpu.make_async_remote_copy(
                src_ref=comm_ref.at[s, H:C, :],
                dst_ref=comm_ref.at[s + 1, H:C, :],
                send_sem=ccw_send.at[s],
                recv_sem=ccw_recv.at[s],
                device_id=(left,),
                device_id_type=pl.DeviceIdType.MESH,
            )
            cw.start()
            ccw.start()
            c_cw = (my - 1 - s) % N_DEV
            c_ccw = (my + 1 + s) % N_DEV
            vt = pblock(c_cw * C, H)
            vb = pblock(c_ccw * C + H, H)
            cw.wait()
            ccw.wait()
            comm_ref[s + 1, 0:H, :] = comm_ref[s + 1, 0:H, :] + vt
            comm_ref[s + 1, H:C, :] = comm_ref[s + 1, H:C, :] + vb

        own_cw = (my + 1) % N_DEV
        own_ccw = (my - 1) % N_DEV
        out_ref[pl.ds(own_cw * C, H), :] = comm_ref[N_DEV - 1, 0:H, :]
        out_ref[pl.ds(own_ccw * C + H, H), :] = comm_ref[N_DEV - 1, H:C, :]

        for t in range(N_DEV - 1):
            k = (N_DEV - 1) + t
            cw = pltpu.make_async_remote_copy(
                src_ref=comm_ref.at[k, 0:H, :],
                dst_ref=comm_ref.at[k + 1, 0:H, :],
                send_sem=cw_send.at[k],
                recv_sem=cw_recv.at[k],
                device_id=(right,),
                device_id_type=pl.DeviceIdType.MESH,
            )
            ccw = pltpu.make_async_remote_copy(
                src_ref=comm_ref.at[k, H:C, :],
                dst_ref=comm_ref.at[k + 1, H:C, :],
                send_sem=ccw_send.at[k],
                recv_sem=ccw_recv.at[k],
                device_id=(left,),
                device_id_type=pl.DeviceIdType.MESH,
            )
            cw.start()
            ccw.start()
            cw.wait()
            ccw.wait()
            c1 = (my - t) % N_DEV
            c2 = (my + t) % N_DEV
            out_ref[pl.ds(c1 * C, H), :] = comm_ref[k + 1, 0:H, :]
            out_ref[pl.ds(c2 * C + H, H), :] = comm_ref[k + 1, H:C, :]

    return pl.pallas_call(
        body,
        out_shape=jax.ShapeDtypeStruct((S, D), jnp.bfloat16),
        in_specs=[
            pl.BlockSpec(memory_space=pltpu.VMEM),
            pl.BlockSpec(memory_space=pltpu.VMEM),
        ],
        out_specs=pl.BlockSpec(memory_space=pltpu.VMEM),
        scratch_shapes=[
            pltpu.VMEM((2 * N_DEV - 1, C, D), jnp.bfloat16),
            pltpu.SemaphoreType.DMA((2 * (N_DEV - 1),)),
            pltpu.SemaphoreType.DMA((2 * (N_DEV - 1),)),
            pltpu.SemaphoreType.DMA((2 * (N_DEV - 1),)),
            pltpu.SemaphoreType.DMA((2 * (N_DEV - 1),)),
        ],
        compiler_params=pltpu.CompilerParams(collective_id=0),
    )(ctx, Wo_l)


def kernel(x, Wq, K_ext, V_ext, Wo):
    B, Sq, d_model = x.shape
    _, Skv, Hl, Dh = K_ext.shape
    Dl = Hl * Dh

    my = lax.axis_index("i")

    xb = x[0].astype(jnp.bfloat16)
    Wq_l = lax.dynamic_slice_in_dim(Wq, my * Dl, Dl, axis=1)
    Q = (xb @ Wq_l.astype(jnp.bfloat16)).reshape(Sq, Hl, Dh)
    K = K_ext[0].astype(jnp.bfloat16)
    V = V_ext[0].astype(jnp.bfloat16)

    n_blk = Sq // BLK
    m = n_blk // PHASES

    def group(t):
        t = t.reshape(m, PHASES, BLK, Hl, Dh)
        return t.transpose(1, 0, 2, 3, 4).reshape(PHASES, m * BLK, Hl, Dh)

    Qg, Kg, Vg = group(Q), group(K), group(V)
    scores = jnp.einsum(
        "pihd,pjhd->phij", Qg, Kg, preferred_element_type=jnp.float32
    ) * SCALE
    w = jax.nn.softmax(scores, axis=-1).astype(jnp.bfloat16)
    ctx = jnp.einsum("phij,pjhd->pihd", w, Vg)
    ctx = (
        ctx.reshape(PHASES, m, BLK, Hl, Dh)
        .transpose(1, 0, 2, 3, 4)
        .reshape(Sq, Dl)
    )

    Wo_l = lax.dynamic_slice_in_dim(Wo, my * Dl, Dl, axis=0)
    out = _wo_allreduce(ctx, Wo_l.astype(jnp.bfloat16))
    return out[None]


# device time: 89244 ns/iter; 2.5898x vs baseline; 1.4380x over previous
import jax
import jax.numpy as jnp
from jax import lax
from jax.experimental import pallas as pl
from jax.experimental.pallas import tpu as pltpu

N_DEV = 4
SCALE = 0.08838834764831843
BLK = 64
PHASES = 4


def _wo_allreduce(ctx, Wo_l):
    S, D = ctx.shape
    C = S // N_DEV
    H = C // 2

    def body(ctx_ref, wo_ref, out_ref, comm_ref, cw_send, cw_recv,
             ccw_send, ccw_recv):
        my = lax.axis_index("i")
        left = (my - 1) % N_DEV
        right = (my + 1) % N_DEV

        barrier_sem = pltpu.get_barrier_semaphore()
        for nbr in [left, right]:
            pl.semaphore_signal(
                barrier_sem, inc=1,
                device_id=(nbr,), device_id_type=pl.DeviceIdType.MESH,
            )
        pl.semaphore_wait(barrier_sem, 2)

        w = wo_ref[...]

        def pblock(row_start, n):
            a = ctx_ref[pl.ds(row_start, n), :]
            return jnp.dot(a, w, preferred_element_type=jnp.float32).astype(
                jnp.bfloat16
            )

        comm_ref[0] = pblock(my * C, C)

        for s in range(N_DEV - 1):
            cw = pltpu.make_async_remote_copy(
                src_ref=comm_ref.at[s, 0:H, :],
                dst_ref=comm_ref.at[s + 1, 0:H, :],
                send_sem=cw_send.at[s],
                recv_sem=cw_recv.at[s],
                device_id=(right,),
                device_id_type=pl.DeviceIdType.MESH,
            )
            ccw = pltpu.make_async_remote_copy(
                src_ref=comm_ref.at[s, H:C, :],
                dst_ref=comm_ref.at[s + 1, H:C, :],
                send_sem=ccw_send.at[s],
                recv_sem=ccw_recv.at[s],
                device_id=(left,),
                device_id_type=pl.DeviceIdType.MESH,
            )
            cw.start()
            ccw.start()
            c_cw = (my - 1 - s) % N_DEV
            c_ccw = (my + 1 + s) % N_DEV
            vt = pblock(c_cw * C, H)
            vb = pblock(c_ccw * C + H, H)
            cw.wait()
            ccw.wait()
            comm_ref[s + 1, 0:H, :] = comm_ref[s + 1, 0:H, :] + vt
            comm_ref[s + 1, H:C, :] = comm_ref[s + 1, H:C, :] + vb

        own_cw = (my + 1) % N_DEV
        own_ccw = (my - 1) % N_DEV
        out_ref[pl.ds(own_cw * C, H), :] = comm_ref[N_DEV - 1, 0:H, :]
        out_ref[pl.ds(own_ccw * C + H, H), :] = comm_ref[N_DEV - 1, H:C, :]

        for t in range(N_DEV - 1):
            k = (N_DEV - 1) + t
            cw = pltpu.make_async_remote_copy(
                src_ref=comm_ref.at[k, 0:H, :],
                dst_ref=comm_ref.at[k + 1, 0:H, :],
                send_sem=cw_send.at[k],
                recv_sem=cw_recv.at[k],
                device_id=(right,),
                device_id_type=pl.DeviceIdType.MESH,
            )
            ccw = pltpu.make_async_remote_copy(
                src_ref=comm_ref.at[k, H:C, :],
                dst_ref=comm_ref.at[k + 1, H:C, :],
                send_sem=ccw_send.at[k],
                recv_sem=ccw_recv.at[k],
                device_id=(left,),
                device_id_type=pl.DeviceIdType.MESH,
            )
            cw.start()
            ccw.start()
            cw.wait()
            ccw.wait()
            c1 = (my - t) % N_DEV
            c2 = (my + t) % N_DEV
            out_ref[pl.ds(c1 * C, H), :] = comm_ref[k + 1, 0:H, :]
            out_ref[pl.ds(c2 * C + H, H), :] = comm_ref[k + 1, H:C, :]

    return pl.pallas_call(
        body,
        out_shape=jax.ShapeDtypeStruct((S, D), jnp.bfloat16),
        in_specs=[
            pl.BlockSpec(memory_space=pltpu.VMEM),
            pl.BlockSpec(memory_space=pltpu.VMEM),
        ],
        out_specs=pl.BlockSpec(memory_space=pltpu.VMEM),
        scratch_shapes=[
            pltpu.VMEM((2 * N_DEV - 1, C, D), jnp.bfloat16),
            pltpu.SemaphoreType.DMA((2 * (N_DEV - 1),)),
            pltpu.SemaphoreType.DMA((2 * (N_DEV - 1),)),
            pltpu.SemaphoreType.DMA((2 * (N_DEV - 1),)),
            pltpu.SemaphoreType.DMA((2 * (N_DEV - 1),)),
        ],
        compiler_params=pltpu.CompilerParams(collective_id=0),
    )(ctx, Wo_l)


def _sparse_attention(x2, Wq_l, K, V):
    S, d_model = x2.shape
    _, Hl, Dh = K.shape
    Dl = Hl * Dh
    n_blk = S // BLK
    m = n_blk // PHASES

    def rows(p, j):
        return (j * PHASES + p) * BLK

    def body(x_ref, wq_ref, k_ref, v_ref, out_ref):
        wq = wq_ref[...]
        for p in range(PHASES):
            xp = jnp.concatenate(
                [x_ref[rows(p, j):rows(p, j) + BLK, :] for j in range(m)]
            ).astype(jnp.bfloat16)
            qp = jnp.dot(
                xp, wq, preferred_element_type=jnp.float32
            ).astype(jnp.bfloat16)
            kp = jnp.concatenate(
                [k_ref[rows(p, j):rows(p, j) + BLK, :, :] for j in range(m)]
            ).astype(jnp.bfloat16)
            vp = jnp.concatenate(
                [v_ref[rows(p, j):rows(p, j) + BLK, :, :] for j in range(m)]
            ).astype(jnp.bfloat16)
            ctx_h = []
            for h in range(Hl):
                qh = qp[:, h * Dh:(h + 1) * Dh]
                s = lax.dot_general(
                    qh, kp[:, h, :],
                    (((1,), (1,)), ((), ())),
                    preferred_element_type=jnp.float32,
                ) * SCALE
                smax = jnp.max(s, axis=-1, keepdims=True)
                e = jnp.exp(s - smax)
                w = (e / jnp.sum(e, axis=-1, keepdims=True)).astype(
                    jnp.bfloat16
                )
                ctx_h.append(
                    jnp.dot(
                        w, vp[:, h, :], preferred_element_type=jnp.float32
                    ).astype(jnp.bfloat16)
                )
            ctx_p = jnp.concatenate(ctx_h, axis=1)
            for j in range(m):
                out_ref[rows(p, j):rows(p, j) + BLK, :] = (
                    ctx_p[j * BLK:(j + 1) * BLK, :]
                )

    return pl.pallas_call(
        body,
        out_shape=jax.ShapeDtypeStruct((S, Dl), jnp.bfloat16),
        in_specs=[pl.BlockSpec(memory_space=pltpu.VMEM)] * 4,
        out_specs=pl.BlockSpec(memory_space=pltpu.VMEM),
    )(x2, Wq_l, K, V)


def kernel(x, Wq, K_ext, V_ext, Wo):
    B, Sq, d_model = x.shape
    _, Skv, Hl, Dh = K_ext.shape
    Dl = Hl * Dh

    my = lax.axis_index("i")

    Wq_l = lax.dynamic_slice_in_dim(Wq, my * Dl, Dl, axis=1)
    ctx = _sparse_attention(
        x[0], Wq_l.astype(jnp.bfloat16), K_ext[0], V_ext[0]
    )

    Wo_l = lax.dynamic_slice_in_dim(Wo, my * Dl, Dl, axis=0)
    out = _wo_allreduce(ctx, Wo_l.astype(jnp.bfloat16))
    return out[None]
